# baseline (speedup 1.0000x reference)
import jax
import jax.numpy as jnp
from jax import lax
from jax.experimental import pallas as pl
from jax.experimental.pallas import tpu as pltpu
from jax.experimental.pallas import tpu_sc as plsc

VOCAB_SIZE = 100000
EMBEDDING_SIZE = 64
BATCH = 4096
SEQ_LEN = 200

NUM_WORKERS = 32
BATCH_PER_W = BATCH // NUM_WORKERS


def _embed_kernel(idx_hbm, table_hbm, pe_hbm, out_hbm,
                  slab, idx_t, pe_v,
                  rows0, rows1, rows2, rows3, ob0, ob1,
                  g0, g1, g2, g3, w0, w1):
    nc = 2
    wid = lax.axis_index("s") * nc + lax.axis_index("c")
    base = wid * BATCH_PER_W

    rows = (rows0, rows1, rows2, rows3)
    gsem = (g0, g1, g2, g3)
    obuf = (ob0, ob1)
    wsem = (w0, w1)

    pltpu.sync_copy(idx_hbm.at[pl.ds(base * SEQ_LEN, BATCH_PER_W * SEQ_LEN)],
                    slab.at[pl.ds(0, BATCH_PER_W * SEQ_LEN)])
    pltpu.sync_copy(pe_hbm, pe_v)

    lanes = lax.iota(jnp.int32, 16)

    sv128 = [(lanes + 16 * k) * BATCH_PER_W for k in range(13)]
    tail = lanes < 8

    def t_body(p, _):
        off = p * SEQ_LEN
        for k in range(12):
            v = slab[pl.ds(off + 16 * k, 16)]
            plsc.store_scatter(idx_t, [sv128[k] + p], v)
        v = slab[pl.ds(off + 192, 16)]
        plsc.store_scatter(idx_t, [sv128[12] + p], v, mask=tail)
        return ()

    lax.fori_loop(0, BATCH_PER_W, t_body, (), unroll=2)

    def g_start(s, b):
        pltpu.async_copy(
            table_hbm.at[idx_t.at[pl.ds(s * BATCH_PER_W, BATCH_PER_W)]],
            rows[b], gsem[b])

    def g_wait(b):
        pltpu.make_async_copy(
            table_hbm.at[idx_t.at[pl.ds(0, BATCH_PER_W)]], rows[b],
            gsem[b]).wait()

    def w_start(s, b):
        for eg in range(8):
            pltpu.async_copy(obuf[b].at[pl.ds(1024 * eg, 1024)],
                             out_hbm.at[s * 8 + eg, wid], wsem[b])

    def w_drain(b):
        for eg in range(8):
            pltpu.make_async_copy(obuf[b].at[pl.ds(1024 * eg, 1024)],
                                  out_hbm.at[eg, wid], wsem[b]).wait()

    ovecs = []
    for k in range(4):
        e = lanes + 16 * k
        ovecs.append((e // 8) * 1024 + (e % 8) * 128)

    def transpose_add(s, rb, b):
        rows_r = rows[rb]
        ob = obuf[b]
        pe4 = [pe_v[s, pl.ds(16 * k, 16)] for k in range(4)]

        def t_loop(t, _):
            for k in range(4):
                v = rows_r[t, pl.ds(16 * k, 16)] + pe4[k]
                plsc.store_scatter(ob, [ovecs[k] + t], v)
            return ()

        lax.fori_loop(0, BATCH_PER_W, t_loop, (), unroll=4)

    for s in range(3):
        g_start(s, s)

    def outer(p, _):
        for q in range(4):
            s = 4 * p + q
            b = q % 2
            g_wait(q)

            @pl.when(s + 3 < SEQ_LEN)
            def _():
                g_start(s + 3, (q + 3) % 4)

            @pl.when(s >= 2)
            def _():
                w_drain(b)

            transpose_add(s, q, b)
            w_start(s, b)
        return ()

    lax.fori_loop(0, SEQ_LEN // 4, outer, ())

    w_drain(0)
    w_drain(1)


@jax.jit
def _run(idx_flat, table, pe):
    mesh = plsc.VectorSubcoreMesh(core_axis_name="c", subcore_axis_name="s")
    fn = pl.kernel(
        _embed_kernel,
        mesh=mesh,
        compiler_params=pltpu.CompilerParams(use_tc_tiling_on_sc=False,
                                             needs_layout_passes=False),
        out_type=jax.ShapeDtypeStruct(
            (SEQ_LEN * 8, NUM_WORKERS, 8 * 128), jnp.float32),
        scratch_types=[
            pltpu.VMEM((BATCH_PER_W * SEQ_LEN + 16,), jnp.int32),
            pltpu.VMEM((SEQ_LEN * BATCH_PER_W,), jnp.int32),
            pltpu.VMEM((SEQ_LEN, EMBEDDING_SIZE), jnp.float32),
        ] + [
            pltpu.VMEM((BATCH_PER_W, EMBEDDING_SIZE), jnp.float32)
            for _ in range(4)
        ] + [
            pltpu.VMEM((8 * 8 * 128,), jnp.float32) for _ in range(2)
        ] + [pltpu.SemaphoreType.DMA for _ in range(6)],
    )
    raw = fn(idx_flat, table, pe)
    out5 = raw.reshape(SEQ_LEN, 8, NUM_WORKERS, 8, 128)
    return out5.transpose(2, 4, 0, 1, 3).reshape(BATCH, SEQ_LEN,
                                                 EMBEDDING_SIZE)


def kernel(inputs, word_embedding, position_embedding):
    idx_flat = inputs.astype(jnp.int32).reshape(BATCH * SEQ_LEN)
    pe = position_embedding[:SEQ_LEN]
    return _run(idx_flat, word_embedding, pe)
